# baseline (speedup 1.0000x reference)
import jax
import jax.numpy as jnp
from jax.experimental import pallas as pl


def _add_kernel(src_ref, jt_ref, pe_ref, out_ref):
    combined = jt_ref[...] + pe_ref[...]
    out_ref[...] = src_ref[...] + combined[None, :, :]


def kernel(src, joint_emb, pos_emb):
    B, D, F, R = src.shape
    W = F * R
    src3 = src.reshape(B, D, W)
    jt = jnp.tile(joint_emb, (1, F))
    pe = pos_emb.reshape(1, W)
    BBLK = 8
    out = pl.pallas_call(
        _add_kernel,
        grid=(B // BBLK,),
        in_specs=[
            pl.BlockSpec((BBLK, D, W), lambda i: (i, 0, 0)),
            pl.BlockSpec((D, W), lambda i: (0, 0)),
            pl.BlockSpec((1, W), lambda i: (0, 0)),
        ],
        out_specs=pl.BlockSpec((BBLK, D, W), lambda i: (i, 0, 0)),
        out_shape=jax.ShapeDtypeStruct((B, D, W), src.dtype),
    )(src3, jt, pe)
    return out.reshape(B, D, F, R)


# speedup vs baseline: 10.5928x; 10.5928x over previous
import jax
import jax.numpy as jnp
from jax.experimental import pallas as pl
from jax.experimental.pallas import tpu as pltpu

_B = 1024
_D = 6
_F = 2048
_R = 6
_BBLK = 512


def _add_kernel(jt_ref, pe_ref, src_ref, out_ref):
    d = pl.program_id(0)
    r = pl.program_id(1)
    add = jt_ref[d, r] + pe_ref[...]
    out_ref[...] = src_ref[...] + add[None, :, :, :]


def kernel(src, joint_emb, pos_emb):
    st = jnp.transpose(src, (1, 3, 0, 2))
    pt = jnp.transpose(pos_emb, (1, 0)).reshape(_R, 1, _F)
    out_t = pl.pallas_call(
        _add_kernel,
        grid=(_D, _R, _B // _BBLK),
        in_specs=[
            pl.BlockSpec(memory_space=pltpu.SMEM),
            pl.BlockSpec((1, 1, _F), lambda d, r, i: (r, 0, 0)),
            pl.BlockSpec((1, 1, _BBLK, _F), lambda d, r, i: (d, r, i, 0)),
        ],
        out_specs=pl.BlockSpec((1, 1, _BBLK, _F), lambda d, r, i: (d, r, i, 0)),
        out_shape=jax.ShapeDtypeStruct((_D, _R, _B, _F), src.dtype),
        compiler_params=pltpu.CompilerParams(
            dimension_semantics=("parallel", "parallel", "parallel"),
        ),
    )(joint_emb, pt, st)
    return jnp.transpose(out_t, (2, 0, 3, 1))


# speedup vs baseline: 10.7291x; 1.0129x over previous
import jax
import jax.numpy as jnp
from jax.experimental import pallas as pl
from jax.experimental.pallas import tpu as pltpu

_B = 1024
_D = 6
_F = 2048
_R = 6
_BBLK = 1024


def _add_kernel(jt_ref, pe_ref, src_ref, out_ref):
    d = pl.program_id(0)
    r = pl.program_id(1)
    add = jt_ref[d, r] + pe_ref[...]
    out_ref[...] = src_ref[...] + add[None, :, :, :]


def kernel(src, joint_emb, pos_emb):
    st = jnp.transpose(src, (1, 3, 0, 2))
    pt = jnp.transpose(pos_emb, (1, 0)).reshape(_R, 1, _F)
    out_t = pl.pallas_call(
        _add_kernel,
        grid=(_D, _R, _B // _BBLK),
        in_specs=[
            pl.BlockSpec(memory_space=pltpu.SMEM),
            pl.BlockSpec((1, 1, _F), lambda d, r, i: (r, 0, 0)),
            pl.BlockSpec((1, 1, _BBLK, _F), lambda d, r, i: (d, r, i, 0)),
        ],
        out_specs=pl.BlockSpec((1, 1, _BBLK, _F), lambda d, r, i: (d, r, i, 0)),
        out_shape=jax.ShapeDtypeStruct((_D, _R, _B, _F), src.dtype),
        compiler_params=pltpu.CompilerParams(
            dimension_semantics=("parallel", "parallel", "parallel"),
        ),
    )(joint_emb, pt, st)
    return jnp.transpose(out_t, (2, 0, 3, 1))
